# Initial kernel scaffold; baseline (speedup 1.0000x reference)
import jax
import jax.numpy as jnp
from jax.experimental import pallas as pl


def kernel(features, edge_index_rsr, edge_index_rtr, edge_index_rur, W_rsr, b_rsr, W_rtr, b_rtr, W_rur, b_rur, W2, b2):
    raise NotImplementedError("write your pallas kernel here")


# speedup vs baseline: 13.0524x; 13.0524x over previous
import functools

import jax
import jax.numpy as jnp
from jax import lax
from jax.experimental import pallas as pl
from jax.experimental.pallas import tpu as pltpu
from jax.experimental.pallas import tpu_sc as plsc

N = 10000
E = 160000
D = 128
HID = 256
C = 16
NP = 10240
EP = 163840
NC, NS = 2, 16
NW = NC * NS

_MESH = plsc.VectorSubcoreMesh(
    core_axis_name="c", subcore_axis_name="s", num_cores=NC, num_subcores=NS)


@functools.partial(
    pl.kernel,
    out_type=jax.ShapeDtypeStruct((6, NP), jnp.float32),
    mesh=_MESH,
    scratch_types=[
        pltpu.VMEM((80, 128), jnp.int32),
        pltpu.VMEM((128,), jnp.float32),
        pltpu.VMEM((640,), jnp.float32),
        pltpu.VMEM_SHARED((3 * NP,), jnp.float32),
    ],
)
def _hist(idx_hbm, out_hbm, idxv, onesv, stage, hist_sh):
    c = lax.axis_index("c")
    s = lax.axis_index("s")
    for k in range(8):
        onesv[pl.ds(k * 16, 16)] = jnp.ones((16,), jnp.float32)
    for k in range(40):
        stage[pl.ds(k * 16, 16)] = jnp.zeros((16,), jnp.float32)
    for hl in range(3):
        pltpu.sync_copy(stage, hist_sh.at[pl.ds(hl * NP + s * 640, 640)])
    plsc.subcore_barrier()
    for hl in range(3):
        pltpu.sync_copy(idx_hbm.at[3 * c + hl, s], idxv)

        def body(j, carry):
            pltpu.sync_copy(onesv, hist_sh.at[idxv.at[j]], add=True)
            return carry

        lax.fori_loop(0, 80, body, 0)
    plsc.subcore_barrier()
    for hl in range(3):
        pltpu.sync_copy(hist_sh.at[pl.ds(hl * NP + s * 640, 640)], stage)
        pltpu.sync_copy(stage, out_hbm.at[3 * c + hl, pl.ds(s * 640, 640)])


@functools.partial(
    pl.kernel,
    out_type=jax.ShapeDtypeStruct((3, NC, NP, D), jnp.float32),
    mesh=_MESH,
    scratch_types=[
        pltpu.VMEM((40, 128), jnp.int32),
        pltpu.VMEM((40, 128), jnp.int32),
        pltpu.VMEM((128, D), jnp.float32),
        pltpu.VMEM((64, D), jnp.float32),
        pltpu.VMEM((64, D), jnp.float32),
        pltpu.VMEM_SHARED((NP, D), jnp.float32),
        pltpu.SemaphoreType.DMA,
    ],
)
def _agg1(xs_hbm, src_hbm, dst_hbm, out_hbm, idxs, idxd, rows, zb, wb,
          acc_sh, sem):
    c = lax.axis_index("c")
    s = lax.axis_index("s")
    wid = c * NS + s

    def zrow(i, carry):
        for k in range(8):
            zb[i, pl.ds(k * 16, 16)] = jnp.zeros((16,), jnp.float32)
        return carry

    lax.fori_loop(0, 64, zrow, 0)

    for e in range(3):
        def zacc(k, carry):
            pltpu.sync_copy(zb, acc_sh.at[pl.ds(s * 640 + k * 64, 64)])
            return carry

        lax.fori_loop(0, 10, zacc, 0)
        plsc.subcore_barrier()
        pltpu.sync_copy(src_hbm.at[e, wid], idxs)
        pltpu.sync_copy(dst_hbm.at[e, wid], idxd)

        def chunk(j, carry):
            pltpu.async_copy(xs_hbm.at[idxs.at[j]], rows, sem).wait()
            pltpu.sync_copy(rows, acc_sh.at[idxd.at[j]], add=True)
            return carry

        lax.fori_loop(0, 40, chunk, 0)
        plsc.subcore_barrier()

        def wback(k, carry):
            pltpu.sync_copy(acc_sh.at[pl.ds(s * 640 + k * 64, 64)], wb)
            pltpu.sync_copy(wb, out_hbm.at[e, c, pl.ds(s * 640 + k * 64, 64)])
            return carry

        lax.fori_loop(0, 10, wback, 0)


@functools.partial(
    pl.kernel,
    out_type=jax.ShapeDtypeStruct((NC, NP, C), jnp.float32),
    mesh=_MESH,
    scratch_types=[
        pltpu.VMEM((120, 128), jnp.int32),
        pltpu.VMEM((120, 128), jnp.int32),
        pltpu.VMEM((128, C), jnp.float32),
        pltpu.VMEM((64, C), jnp.float32),
        pltpu.VMEM((640, C), jnp.float32),
        pltpu.VMEM_SHARED((NP, C), jnp.float32),
        pltpu.SemaphoreType.DMA,
    ],
    compiler_params=pltpu.CompilerParams(use_tc_tiling_on_sc=False),
)
def _agg2(tab_hbm, src_hbm, dst_hbm, out_hbm, idxs, idxd, rows, zb, wb,
          acc_sh, sem):
    c = lax.axis_index("c")
    s = lax.axis_index("s")
    wid = c * NS + s

    def zrow(i, carry):
        zb[i, :] = jnp.zeros((16,), jnp.float32)
        return carry

    lax.fori_loop(0, 64, zrow, 0)

    def zacc(k, carry):
        pltpu.sync_copy(zb, acc_sh.at[pl.ds(s * 640 + k * 64, 64)])
        return carry

    lax.fori_loop(0, 10, zacc, 0)
    plsc.subcore_barrier()
    pltpu.sync_copy(src_hbm.at[wid], idxs)
    pltpu.sync_copy(dst_hbm.at[wid], idxd)

    def chunk(j, carry):
        pltpu.async_copy(tab_hbm.at[idxs.at[j]], rows, sem).wait()
        pltpu.sync_copy(rows, acc_sh.at[idxd.at[j]], add=True)
        return carry

    lax.fori_loop(0, 120, chunk, 0)
    plsc.subcore_barrier()
    pltpu.sync_copy(acc_sh.at[pl.ds(s * 640, 640)], wb)
    pltpu.sync_copy(wb, out_hbm.at[c, pl.ds(s * 640, 640)])


def _prep_body(hist_ref, feat_ref, xs_ref, nd3_ref, nsa_ref, nda_ref):
    h = hist_ref[:]
    feat = feat_ref[:]
    for e in range(3):
        ns_e = lax.rsqrt(h[:, 2 * e:2 * e + 1] + 1.0)
        xs_ref[e] = feat * ns_e
        nd3_ref[:, e:e + 1] = lax.rsqrt(h[:, 2 * e + 1:2 * e + 2] + 1.0)
    nsa_ref[:] = lax.rsqrt(h[:, 0:1] + h[:, 2:3] + h[:, 4:5] + 1.0)
    nda_ref[:] = lax.rsqrt(h[:, 1:2] + h[:, 3:4] + h[:, 5:6] + 1.0)


_prep = pl.pallas_call(
    _prep_body,
    grid=(NP // 256,),
    in_specs=[
        pl.BlockSpec((256, 6), lambda i: (i, 0)),
        pl.BlockSpec((256, D), lambda i: (i, 0)),
    ],
    out_specs=[
        pl.BlockSpec((3, 256, D), lambda i: (0, i, 0)),
        pl.BlockSpec((256, 3), lambda i: (i, 0)),
        pl.BlockSpec((256, 1), lambda i: (i, 0)),
        pl.BlockSpec((256, 1), lambda i: (i, 0)),
    ],
    out_shape=[
        jax.ShapeDtypeStruct((3, NP, D), jnp.float32),
        jax.ShapeDtypeStruct((NP, 3), jnp.float32),
        jax.ShapeDtypeStruct((NP, 1), jnp.float32),
        jax.ShapeDtypeStruct((NP, 1), jnp.float32),
    ],
)


def _mm_body(p_ref, xs_ref, nd3_ref, nsa_ref, w_ref, bbar_ref, w2_ref,
             out_ref):
    acc = jnp.zeros((256, HID), jnp.float32)
    for e in range(3):
        a = p_ref[e, 0] + p_ref[e, 1] + xs_ref[e]
        a = a * nd3_ref[:, e:e + 1]
        acc += jnp.dot(a, w_ref[e], preferred_element_type=jnp.float32)
    h = jnp.maximum(acc * (1.0 / 3.0) + bbar_ref[:], 0.0)
    hw = jnp.dot(h, w2_ref[:], preferred_element_type=jnp.float32)
    out_ref[:] = hw * nsa_ref[:]


_mm = pl.pallas_call(
    _mm_body,
    grid=(NP // 256,),
    in_specs=[
        pl.BlockSpec((3, NC, 256, D), lambda i: (0, 0, i, 0)),
        pl.BlockSpec((3, 256, D), lambda i: (0, i, 0)),
        pl.BlockSpec((256, 3), lambda i: (i, 0)),
        pl.BlockSpec((256, 1), lambda i: (i, 0)),
        pl.BlockSpec((3, D, HID), lambda i: (0, 0, 0)),
        pl.BlockSpec((1, HID), lambda i: (0, 0)),
        pl.BlockSpec((HID, C), lambda i: (0, 0)),
    ],
    out_specs=pl.BlockSpec((256, C), lambda i: (i, 0)),
    out_shape=jax.ShapeDtypeStruct((NP, C), jnp.float32),
)


def _final_body(a2_ref, hws_ref, nda_ref, b2_ref, out_ref):
    agg = a2_ref[0] + a2_ref[1] + hws_ref[:]
    out_ref[:] = agg * nda_ref[:] + b2_ref[:]


_final = pl.pallas_call(
    _final_body,
    grid=(NP // 256,),
    in_specs=[
        pl.BlockSpec((NC, 256, C), lambda i: (0, i, 0)),
        pl.BlockSpec((256, C), lambda i: (i, 0)),
        pl.BlockSpec((256, 1), lambda i: (i, 0)),
        pl.BlockSpec((1, C), lambda i: (0, 0)),
    ],
    out_specs=pl.BlockSpec((256, C), lambda i: (i, 0)),
    out_shape=jax.ShapeDtypeStruct((NP, C), jnp.float32),
)


def kernel(features, edge_index_rsr, edge_index_rtr, edge_index_rur,
           W_rsr, b_rsr, W_rtr, b_rtr, W_rur, b_rur, W2, b2):
    srcs = [edge_index_rsr[0].astype(jnp.int32),
            edge_index_rtr[0].astype(jnp.int32),
            edge_index_rur[0].astype(jnp.int32)]
    dsts = [edge_index_rsr[1].astype(jnp.int32),
            edge_index_rtr[1].astype(jnp.int32),
            edge_index_rur[1].astype(jnp.int32)]
    pad = jnp.full((EP - E,), N, jnp.int32)
    sp = [jnp.concatenate([x, pad]) for x in srcs]
    dp = [jnp.concatenate([x, pad]) for x in dsts]

    idx6 = jnp.stack([sp[0], dp[0], sp[1], dp[1], sp[2], dp[2]])
    off = (jnp.arange(6, dtype=jnp.int32) % 3) * NP
    idx6_off = (idx6 + off[:, None]).reshape(6, NS, 80, 128)
    srcg = jnp.stack([sp[e] + e * NP for e in range(3)]).reshape(
        3, NW, 40, 128)
    dst3 = jnp.stack(dp).reshape(3, NW, 40, 128)
    src2 = jnp.concatenate(sp).reshape(NW, 120, 128)
    dst2 = jnp.concatenate(dp).reshape(NW, 120, 128)
    featp = jnp.pad(features, ((0, NP - N), (0, 0)))

    hist6 = _hist(idx6_off)
    xs3, nd3, nsa, nda = _prep(hist6.T, featp)
    p = _agg1(xs3.reshape(3 * NP, D), srcg, dst3)
    wst = jnp.stack([W_rsr, W_rtr, W_rur])
    bbar = ((b_rsr + b_rtr + b_rur) * (1.0 / 3.0)).reshape(1, HID)
    hws = _mm(p, xs3, nd3, nsa, wst, bbar, W2)
    a2 = _agg2(hws, src2, dst2)
    outp = _final(a2, hws, nda, b2.reshape(1, C))
    return outp[:N]
